# baseline (speedup 1.0000x reference)
import jax
import jax.numpy as jnp
from jax.experimental import pallas as pl
from jax.experimental.pallas import tpu as pltpu

B, N = 8, 2810
IN, HID, LAT, OUT = 256, 128, 64, 128
KDIM = N * LAT
KBLK = 35968
KSTEPS = KDIM // KBLK
RCH = 352
ADJ_CH = 704
NADJ = (N + ADJ_CH - 1) // ADJ_CH


def _gcn_body(x_ref, adj_hbm, w1_ref, w2_ref, out_ref,
              t_ref, t2_ref, adj_ref, sems):
    i = pl.program_id(0)

    @pl.when(i == 0)
    def _start_adj():
        for c in range(NADJ):
            r0 = c * ADJ_CH
            cr = min(ADJ_CH, N - r0)
            pltpu.make_async_copy(
                adj_hbm.at[r0:r0 + cr, :],
                adj_ref.at[r0:r0 + cr, :],
                sems.at[c],
            ).start()

    @pl.when(i < B)
    def _tphase():
        t = jnp.dot(x_ref[0], w1_ref[...], preferred_element_type=jnp.float32)
        for bb in range(B):
            @pl.when(i == bb)
            def _store():
                t_ref[:, bb * HID:(bb + 1) * HID] = t.astype(jnp.bfloat16)

    @pl.when(i == B)
    def _hops():
        for c in range(NADJ):
            r0 = c * ADJ_CH
            cr = min(ADJ_CH, N - r0)
            pltpu.make_async_copy(
                adj_hbm.at[r0:r0 + cr, :],
                adj_ref.at[r0:r0 + cr, :],
                sems.at[c],
            ).wait()
        w2 = w2_ref[...]
        for r0 in range(0, N, RCH):
            cr = min(RCH, N - r0)
            a_bf = adj_ref[r0:r0 + cr, :].astype(jnp.bfloat16)
            h1_r = jnp.maximum(
                jnp.dot(a_bf, t_ref[...],
                        preferred_element_type=jnp.float32), 0.0)
            for bb in range(B):
                t2_ref[r0:r0 + cr, bb * LAT:(bb + 1) * LAT] = jnp.dot(
                    h1_r[:, bb * HID:(bb + 1) * HID], w2,
                    preferred_element_type=jnp.float32).astype(jnp.bfloat16)
        for r0 in range(0, N, RCH):
            cr = min(RCH, N - r0)
            a_bf = adj_ref[r0:r0 + cr, :].astype(jnp.bfloat16)
            out_ref[r0:r0 + cr, :] = jnp.maximum(
                jnp.dot(a_bf, t2_ref[...],
                        preferred_element_type=jnp.float32), 0.0)


def _fc_body(flat_ref, w_ref, b_ref, out_ref):
    k = pl.program_id(0)
    p = jnp.dot(flat_ref[...], w_ref[...], preferred_element_type=jnp.float32)

    @pl.when(k == 0)
    def _init():
        out_ref[...] = p + b_ref[...]

    @pl.when(k != 0)
    def _acc():
        out_ref[...] += p


def _fc_call(flat, W, bvec):
    return pl.pallas_call(
        _fc_body,
        grid=(KSTEPS,),
        in_specs=[
            pl.BlockSpec((B, KBLK), lambda k: (0, k)),
            pl.BlockSpec((KBLK, OUT), lambda k: (k, 0)),
            pl.BlockSpec((1, OUT), lambda k: (0, 0)),
        ],
        out_specs=pl.BlockSpec((B, OUT), lambda k: (0, 0)),
        out_shape=jax.ShapeDtypeStruct((B, OUT), jnp.float32),
        compiler_params=pltpu.CompilerParams(
            vmem_limit_bytes=60 * 1024 * 1024,
        ),
    )(flat, W, bvec.reshape(1, OUT))


@jax.jit
def kernel(x, adj, W1, W2, FCm_W, FCm_b, FCv_W, FCv_b):
    h2t = pl.pallas_call(
        _gcn_body,
        grid=(B + 1,),
        in_specs=[
            pl.BlockSpec((1, N, IN), lambda i: (jnp.minimum(i, B - 1), 0, 0)),
            pl.BlockSpec(memory_space=pl.ANY),
            pl.BlockSpec((IN, HID), lambda i: (0, 0)),
            pl.BlockSpec((HID, LAT), lambda i: (0, 0)),
        ],
        out_specs=pl.BlockSpec((N, B * LAT), lambda i: (0, 0)),
        out_shape=jax.ShapeDtypeStruct((N, B * LAT), jnp.float32),
        scratch_shapes=[
            pltpu.VMEM((N, B * HID), jnp.bfloat16),
            pltpu.VMEM((N, B * LAT), jnp.bfloat16),
            pltpu.VMEM((N, N), jnp.float32),
            pltpu.SemaphoreType.DMA((NADJ,)),
        ],
        compiler_params=pltpu.CompilerParams(
            vmem_limit_bytes=62 * 1024 * 1024,
        ),
    )(x, adj, W1, W2)

    flat = h2t.reshape(N, B, LAT).transpose(1, 0, 2).reshape(B, KDIM)
    mean = _fc_call(flat, FCm_W, FCm_b)
    log_var = _fc_call(flat, FCv_W, FCv_b)
    return (mean, log_var)


# speedup vs baseline: 1.2329x; 1.2329x over previous
import jax
import jax.numpy as jnp
from jax.experimental import pallas as pl
from jax.experimental.pallas import tpu as pltpu

B, N = 8, 2810
IN, HID, LAT, OUT = 256, 128, 64, 128
KDIM = N * LAT
KBLK = 35968
KSTEPS = KDIM // KBLK
RCH = 352
ADJ_CH = 704
NADJ = (N + ADJ_CH - 1) // ADJ_CH


def _gcn_body(x_ref, adj_hbm, w1_ref, w2_ref, out_ref,
              t_ref, t2_ref, adj_ref, sems):
    i = pl.program_id(0)

    @pl.when(i == 0)
    def _start_adj():
        for c in range(NADJ):
            r0 = c * ADJ_CH
            cr = min(ADJ_CH, N - r0)
            pltpu.make_async_copy(
                adj_hbm.at[r0:r0 + cr, :],
                adj_ref.at[r0:r0 + cr, :],
                sems.at[c],
            ).start()

    @pl.when(i < B)
    def _tphase():
        t = jnp.dot(x_ref[0], w1_ref[...], preferred_element_type=jnp.float32)
        for bb in range(B):
            @pl.when(i == bb)
            def _store():
                t_ref[:, bb * HID:(bb + 1) * HID] = t.astype(jnp.bfloat16)

    @pl.when(i == B)
    def _hops():
        for c in range(NADJ):
            r0 = c * ADJ_CH
            cr = min(ADJ_CH, N - r0)
            pltpu.make_async_copy(
                adj_hbm.at[r0:r0 + cr, :],
                adj_ref.at[r0:r0 + cr, :],
                sems.at[c],
            ).wait()
        w2 = w2_ref[...]
        for r0 in range(0, N, RCH):
            cr = min(RCH, N - r0)
            a_bf = adj_ref[r0:r0 + cr, :].astype(jnp.bfloat16)
            h1_r = jnp.maximum(
                jnp.dot(a_bf, t_ref[...],
                        preferred_element_type=jnp.float32), 0.0)
            for bb in range(B):
                t2_ref[r0:r0 + cr, bb * LAT:(bb + 1) * LAT] = jnp.dot(
                    h1_r[:, bb * HID:(bb + 1) * HID], w2,
                    preferred_element_type=jnp.float32).astype(jnp.bfloat16)
        for r0 in range(0, N, RCH):
            cr = min(RCH, N - r0)
            a_bf = adj_ref[r0:r0 + cr, :].astype(jnp.bfloat16)
            h2_r = jnp.maximum(
                jnp.dot(a_bf, t2_ref[...],
                        preferred_element_type=jnp.float32), 0.0
            ).astype(jnp.bfloat16)
            for bb in range(B):
                out_ref[bb, r0:r0 + cr, :] = h2_r[:, bb * LAT:(bb + 1) * LAT]


def _fc_body(flat_ref, w_ref, b_ref, out_ref):
    k = pl.program_id(0)
    p = jnp.dot(flat_ref[...].astype(jnp.float32), w_ref[...],
                preferred_element_type=jnp.float32)

    @pl.when(k == 0)
    def _init():
        out_ref[...] = p + b_ref[...]

    @pl.when(k != 0)
    def _acc():
        out_ref[...] += p


def _fc_call(flat, W, bvec):
    return pl.pallas_call(
        _fc_body,
        grid=(KSTEPS,),
        in_specs=[
            pl.BlockSpec((B, KBLK), lambda k: (0, k)),
            pl.BlockSpec((KBLK, OUT), lambda k: (k, 0)),
            pl.BlockSpec((1, OUT), lambda k: (0, 0)),
        ],
        out_specs=pl.BlockSpec((B, OUT), lambda k: (0, 0)),
        out_shape=jax.ShapeDtypeStruct((B, OUT), jnp.float32),
        compiler_params=pltpu.CompilerParams(
            vmem_limit_bytes=60 * 1024 * 1024,
        ),
    )(flat, W, bvec.reshape(1, OUT))


@jax.jit
def kernel(x, adj, W1, W2, FCm_W, FCm_b, FCv_W, FCv_b):
    h2t = pl.pallas_call(
        _gcn_body,
        grid=(B + 1,),
        in_specs=[
            pl.BlockSpec((1, N, IN), lambda i: (jnp.minimum(i, B - 1), 0, 0)),
            pl.BlockSpec(memory_space=pl.ANY),
            pl.BlockSpec((IN, HID), lambda i: (0, 0)),
            pl.BlockSpec((HID, LAT), lambda i: (0, 0)),
        ],
        out_specs=pl.BlockSpec((B, N, LAT), lambda i: (0, 0, 0)),
        out_shape=jax.ShapeDtypeStruct((B, N, LAT), jnp.bfloat16),
        scratch_shapes=[
            pltpu.VMEM((N, B * HID), jnp.bfloat16),
            pltpu.VMEM((N, B * LAT), jnp.bfloat16),
            pltpu.VMEM((N, N), jnp.float32),
            pltpu.SemaphoreType.DMA((NADJ,)),
        ],
        compiler_params=pltpu.CompilerParams(
            vmem_limit_bytes=62 * 1024 * 1024,
        ),
    )(x, adj, W1, W2)

    flat = h2t.reshape(B, KDIM)
    mean = _fc_call(flat, FCm_W, FCm_b)
    log_var = _fc_call(flat, FCv_W, FCv_b)
    return (mean, log_var)
